# Initial kernel scaffold; baseline (speedup 1.0000x reference)
import jax
import jax.numpy as jnp
from jax.experimental import pallas as pl


def kernel(x, segment_ids, W, b):
    raise NotImplementedError("write your pallas kernel here")


# speedup vs baseline: 3.9429x; 3.9429x over previous
import functools

import jax
import jax.numpy as jnp
from jax.experimental import pallas as pl

B, NF, Fm, H, NC = 8, 4096, 8, 128, 512
FmH = Fm * H
NFB = 4
BLK = NF // NFB


def _scores_body(x_ref, w_ref, o_ref):
    xb = x_ref[0]
    res = jax.lax.dot_general(xb, w_ref[...], (((1,), (0,)), ((), ())),
                              preferred_element_type=jnp.float32)
    o_ref[...] = res.reshape(1, 1, BLK)


def _weights_body(s_ref, seg_ref, o_ref):
    s = s_ref[:, 0, :]
    m = jnp.max(s, axis=1, keepdims=True)
    ex = jnp.exp(s - m)
    seg = seg_ref[0, 0]
    cols = jax.lax.broadcasted_iota(jnp.int32, (NF, NC), 1)
    onehot = (cols == seg[:, None]).astype(jnp.float32)
    denom = jax.lax.dot_general(ex, onehot, (((1,), (0,)), ((), ())),
                                preferred_element_type=jnp.float32)
    denom_g = jax.lax.dot_general(denom, onehot, (((1,), (1,)), ((), ())),
                                  preferred_element_type=jnp.float32)
    o_ref[...] = (ex / denom_g).reshape(B, 1, NF)


def _pool_body(x_ref, w_ref, seg_ref, o_ref):
    nfb = pl.program_id(1)
    seg = seg_ref[0, 0]
    cols = jax.lax.broadcasted_iota(jnp.int32, (BLK, NC), 1)
    onehot = (cols == seg[:, None]).astype(jnp.float32)
    a = onehot * w_ref[0, 0][:, None]
    part = jax.lax.dot_general(a, x_ref[0], (((0,), (0,)), ((), ())),
                               preferred_element_type=jnp.float32)

    @pl.when(nfb == 0)
    def _():
        o_ref[0] = part

    @pl.when(nfb != 0)
    def _():
        o_ref[0] += part


@jax.jit
def kernel(x, segment_ids, W, b):
    del b
    xm = x.reshape(B, NF, FmH)
    seg2d = segment_ids.astype(jnp.int32).reshape(1, 1, NF)
    wfull = (jnp.tile(W[:, 0], Fm) / Fm).reshape(FmH, 1)

    scores = pl.pallas_call(
        _scores_body,
        grid=(B, NFB),
        in_specs=[
            pl.BlockSpec((1, BLK, FmH), lambda bi, ni: (bi, ni, 0)),
            pl.BlockSpec((FmH, 1), lambda bi, ni: (0, 0)),
        ],
        out_specs=pl.BlockSpec((1, 1, BLK), lambda bi, ni: (bi, 0, ni)),
        out_shape=jax.ShapeDtypeStruct((B, 1, NF), jnp.float32),
    )(xm, wfull)

    w = pl.pallas_call(
        _weights_body,
        in_specs=[
            pl.BlockSpec((B, 1, NF), lambda: (0, 0, 0)),
            pl.BlockSpec((1, 1, NF), lambda: (0, 0, 0)),
        ],
        out_specs=pl.BlockSpec((B, 1, NF), lambda: (0, 0, 0)),
        out_shape=jax.ShapeDtypeStruct((B, 1, NF), jnp.float32),
    )(scores, seg2d)

    pooled = pl.pallas_call(
        _pool_body,
        grid=(B, NFB),
        in_specs=[
            pl.BlockSpec((1, BLK, FmH), lambda bi, ni: (bi, ni, 0)),
            pl.BlockSpec((1, 1, BLK), lambda bi, ni: (bi, 0, ni)),
            pl.BlockSpec((1, 1, BLK), lambda bi, ni: (0, 0, ni)),
        ],
        out_specs=pl.BlockSpec((1, NC, FmH), lambda bi, ni: (bi, 0, 0)),
        out_shape=jax.ShapeDtypeStruct((B, NC, FmH), jnp.float32),
    )(xm, w, seg2d)

    return pooled.reshape(B, NC, Fm, H)
